# baseline (speedup 1.0000x reference)
import functools

import jax
import jax.numpy as jnp
from jax import lax
from jax.experimental import pallas as pl
from jax.experimental.pallas import tpu as pltpu
from jax.experimental.pallas import tpu_sc as plsc

D = 64
CHUNK = 128
K = 5


@functools.cache
def _make_kernel(n_blocks_per_w: int, nc: int, ns: int):
    nw = nc * ns
    n_blocks = nw * n_blocks_per_w
    n_chunks_per_w = n_blocks_per_w * K
    mesh = plsc.VectorSubcoreMesh(core_axis_name="c", subcore_axis_name="s")
    assert n_blocks_per_w % 2 == 0

    @functools.partial(
        pl.kernel,
        out_type=jax.ShapeDtypeStruct((n_blocks * K, CHUNK, D), jnp.float32),
        mesh=mesh,
        scratch_types=[
            pltpu.VMEM((n_chunks_per_w, CHUNK), jnp.int32),
            pltpu.VMEM((2, K, CHUNK, D), jnp.float32),
            pltpu.SemaphoreType.DMA,
            pltpu.SemaphoreType.DMA,
            pltpu.SemaphoreType.DMA,
            pltpu.SemaphoreType.DMA,
        ],
        compiler_params=pltpu.CompilerParams(use_tc_tiling_on_sc=False),
    )
    def emb_kernel(idx_hbm, w_hbm, out_hbm, idx_v, rows_v, ga, gb, oa, ob):
        wid = lax.axis_index("s") * nc + lax.axis_index("c")
        chunk_base = wid * n_chunks_per_w

        pltpu.sync_copy(idx_hbm.at[wid], idx_v)

        def fire_gathers(blk, buf, sem):
            return [
                pltpu.async_copy(
                    w_hbm.at[idx_v.at[blk * K + j]], rows_v.at[buf, j], sem)
                for j in range(K)
            ]

        def fire_wb(blk, buf, sem):
            return pltpu.async_copy(
                rows_v.at[buf], out_hbm.at[pl.ds(chunk_base + blk * K, K)], sem)

        @pl.loop(0, n_blocks_per_w, step=2)
        def _(b0):
            ha = fire_gathers(b0, 0, ga)
            hb = fire_gathers(b0 + 1, 1, gb)
            for h in ha:
                h.wait()
            wa = fire_wb(b0, 0, oa)
            for h in hb:
                h.wait()
            wb = fire_wb(b0 + 1, 1, ob)
            wa.wait()
            wb.wait()

    return emb_kernel


def kernel(token_ids, weight):
    b, h = token_ids.shape
    n = b * h
    info = plsc.get_sparse_core_info()
    nc, ns = info.num_cores, info.num_subcores
    nw = nc * ns
    rows_per_block = K * CHUNK
    assert n % (nw * rows_per_block) == 0
    n_blocks_per_w = n // (nw * rows_per_block)
    idx = token_ids.reshape(nw, n_blocks_per_w * K, CHUNK).astype(jnp.int32)
    out = _make_kernel(n_blocks_per_w, nc, ns)(idx, weight)
    return out.reshape(b, h, D)
